# baseline (speedup 1.0000x reference)
import functools
import math

import jax
import jax.numpy as jnp
from jax import lax
from jax.experimental import pallas as pl
from jax.experimental.pallas import tpu as pltpu
from jax.experimental.pallas import tpu_sc as plsc

_N = 60
_F = 32
_B = 4096
_T = 50
_NC, _NS = 2, 16
_NW = _NC * _NS
_CPW = _B // _NW
_LANES = 16
_NG = _CPW // _LANES
_BBLK = 1024


def _sc_histogram_body(ids_hbm, out_hbm, ids_v, cnt_v, in_sem):
    wid = lax.axis_index("s") * _NC + lax.axis_index("c")
    lanes = lax.broadcasted_iota(jnp.int32, (_LANES,), 0)
    zeros = jnp.zeros((_LANES,), jnp.float32)
    ones = jnp.ones((_LANES,), jnp.float32)
    col0 = wid * _CPW
    in_copy = pltpu.async_copy(ids_hbm.at[:, pl.ds(col0, _CPW)], ids_v, in_sem)

    @plsc.parallel_loop(0, _N, step=1, unroll=2)
    def _zero_row(r):
        for c in range(0, _CPW, _LANES):
            cnt_v[r, pl.ds(c, _LANES)] = zeros

    in_copy.wait()

    @plsc.parallel_loop(0, _T, step=1, unroll=4)
    def _step(t):
        tv = lanes * 0 + t
        for g in range(_NG):
            cols = g * _LANES + lanes
            tok = plsc.load_gather(ids_v, [tv, cols])
            plsc.addupdate_scatter(cnt_v, [tok, cols], ones)

    pltpu.sync_copy(cnt_v, out_hbm.at[:, pl.ds(col0, _CPW)])


@functools.cache
def _sc_histogram():
    mesh = plsc.VectorSubcoreMesh(
        core_axis_name="c", subcore_axis_name="s", num_cores=_NC, num_subcores=_NS
    )
    return pl.kernel(
        _sc_histogram_body,
        out_type=jax.ShapeDtypeStruct((_N, _B), jnp.float32),
        mesh=mesh,
        scratch_types=[
            pltpu.VMEM((_T, _CPW), jnp.int32),
            pltpu.VMEM((_N, _CPW), jnp.float32),
            pltpu.SemaphoreType.DMA,
        ],
        compiler_params=pltpu.CompilerParams(
            needs_layout_passes=False, disable_bounds_checks=True),
    )


def _tables_body(rl_ref, w1_ref, w2_ref):
    rlt = jnp.transpose(rl_ref[...])
    m = jnp.max(rlt, axis=0, keepdims=True)
    e = jnp.exp(rlt - m)
    pt = e / jnp.sum(e, axis=0, keepdims=True)
    fi = lax.broadcasted_iota(jnp.int32, (_F, _N), 0)
    ki = lax.broadcasted_iota(jnp.int32, (_F, _N), 1)
    thf = ((fi * ki) % _N).astype(jnp.float32) * (2.0 * math.pi / _N)
    re = jnp.dot(jnp.cos(thf), pt, preferred_element_type=jnp.float32,
                 precision=lax.Precision.HIGHEST)
    im = -jnp.dot(jnp.sin(thf), pt, preferred_element_type=jnp.float32,
                  precision=lax.Precision.HIGHEST)
    live = fi < (_N // 2 + 1)
    lam = jnp.where(
        live, 0.5 * jnp.log(jnp.maximum(re * re + im * im, 1e-30)), 0.0)
    alp = jnp.where(live, jnp.arctan2(im, re), 0.0)
    w1_ref[...] = jnp.concatenate([lam, alp], axis=0)
    mi = lax.broadcasted_iota(jnp.int32, (_N, _F), 0)
    fj = lax.broadcasted_iota(jnp.int32, (_N, _F), 1)
    thi = ((mi * fj) % _N).astype(jnp.float32) * (2.0 * math.pi / _N)
    w = jnp.where((fj == 0) | (fj == _N // 2), 1.0, 2.0) * (1.0 / _N)
    w = jnp.where(fj < (_N // 2 + 1), w, 0.0)
    w2_ref[...] = jnp.concatenate([w * jnp.cos(thi), -w * jnp.sin(thi)], axis=1)


_tc_tables = pl.pallas_call(
    _tables_body,
    out_shape=[
        jax.ShapeDtypeStruct((2 * _F, _N), jnp.float32),
        jax.ShapeDtypeStruct((_N, 2 * _F), jnp.float32),
    ],
)


def _scan_body(w1_ref, w2_ref, cnt_ref, out_ref):
    la = jnp.dot(w1_ref[...], cnt_ref[...], preferred_element_type=jnp.float32)
    l = la[:_F, :]
    a = la[_F:, :]
    el = jnp.exp(l)
    s2 = jnp.concatenate([el * jnp.cos(a), el * jnp.sin(a)], axis=0)
    s = jnp.dot(w2_ref[...], s2, preferred_element_type=jnp.float32)
    out_ref[...] = jnp.log(jnp.maximum(s, 1e-9))


_tc_scan = pl.pallas_call(
    _scan_body,
    grid=(_B // _BBLK,),
    in_specs=[
        pl.BlockSpec((2 * _F, _N), lambda i: (0, 0)),
        pl.BlockSpec((_N, 2 * _F), lambda i: (0, 0)),
        pl.BlockSpec((_N, _BBLK), lambda i: (0, i)),
    ],
    out_specs=pl.BlockSpec((_N, _BBLK), lambda i: (0, i)),
    out_shape=jax.ShapeDtypeStruct((_N, _B), jnp.float32),
)


def kernel(route_logits, input_ids, mul):
    del mul
    counts_t = _sc_histogram()(jnp.transpose(input_ids))
    w1, w2 = _tc_tables(route_logits)
    return jnp.transpose(_tc_scan(w1, w2, counts_t))


# speedup vs baseline: 1.0366x; 1.0366x over previous
import functools
import math

import jax
import jax.numpy as jnp
from jax import lax
from jax.experimental import pallas as pl
from jax.experimental.pallas import tpu as pltpu
from jax.experimental.pallas import tpu_sc as plsc

_N = 60
_F = 32
_B = 4096
_T = 50
_NC, _NS = 2, 16
_NW = _NC * _NS
_CPW = _B // _NW
_LANES = 16
_NG = _CPW // _LANES
_BBLK = 2048


def _sc_histogram_body(ids_hbm, out_hbm, ids_v, cnt_v, in_sem):
    wid = lax.axis_index("s") * _NC + lax.axis_index("c")
    lanes = lax.broadcasted_iota(jnp.int32, (_LANES,), 0)
    zeros = jnp.zeros((_LANES,), jnp.float32)
    ones = jnp.ones((_LANES,), jnp.float32)
    col0 = wid * _CPW
    in_copy = pltpu.async_copy(ids_hbm.at[:, pl.ds(col0, _CPW)], ids_v, in_sem)

    @plsc.parallel_loop(0, _N, step=1, unroll=2)
    def _zero_row(r):
        for c in range(0, _CPW, _LANES):
            cnt_v[r, pl.ds(c, _LANES)] = zeros

    in_copy.wait()

    @plsc.parallel_loop(0, _T, step=1, unroll=2)
    def _step(t):
        tv = lanes * 0 + t
        for g in range(_NG):
            cols = g * _LANES + lanes
            tok = plsc.load_gather(ids_v, [tv, cols])
            plsc.addupdate_scatter(cnt_v, [tok, cols], ones)

    pltpu.sync_copy(cnt_v, out_hbm.at[:, pl.ds(col0, _CPW)])


@functools.cache
def _sc_histogram():
    mesh = plsc.VectorSubcoreMesh(
        core_axis_name="c", subcore_axis_name="s", num_cores=_NC, num_subcores=_NS
    )
    return pl.kernel(
        _sc_histogram_body,
        out_type=jax.ShapeDtypeStruct((_N, _B), jnp.float32),
        mesh=mesh,
        scratch_types=[
            pltpu.VMEM((_T, _CPW), jnp.int32),
            pltpu.VMEM((_N, _CPW), jnp.float32),
            pltpu.SemaphoreType.DMA,
        ],
        compiler_params=pltpu.CompilerParams(
            needs_layout_passes=False, disable_bounds_checks=True),
    )


def _tables_body(rl_ref, w1_ref, w2_ref):
    rlt = jnp.transpose(rl_ref[...])
    m = jnp.max(rlt, axis=0, keepdims=True)
    e = jnp.exp(rlt - m)
    pt = e / jnp.sum(e, axis=0, keepdims=True)
    fi = lax.broadcasted_iota(jnp.int32, (_F, _N), 0)
    ki = lax.broadcasted_iota(jnp.int32, (_F, _N), 1)
    thf = ((fi * ki) % _N).astype(jnp.float32) * (2.0 * math.pi / _N)
    re = jnp.dot(jnp.cos(thf), pt, preferred_element_type=jnp.float32,
                 precision=lax.Precision.HIGHEST)
    im = -jnp.dot(jnp.sin(thf), pt, preferred_element_type=jnp.float32,
                  precision=lax.Precision.HIGHEST)
    live = fi < (_N // 2 + 1)
    lam = jnp.where(
        live, 0.5 * jnp.log(jnp.maximum(re * re + im * im, 1e-30)), 0.0)
    alp = jnp.where(live, jnp.arctan2(im, re), 0.0)
    w1_ref[...] = jnp.concatenate([lam, alp], axis=0)
    mi = lax.broadcasted_iota(jnp.int32, (_N, _F), 0)
    fj = lax.broadcasted_iota(jnp.int32, (_N, _F), 1)
    thi = ((mi * fj) % _N).astype(jnp.float32) * (2.0 * math.pi / _N)
    w = jnp.where((fj == 0) | (fj == _N // 2), 1.0, 2.0) * (1.0 / _N)
    w = jnp.where(fj < (_N // 2 + 1), w, 0.0)
    w2_ref[...] = jnp.concatenate([w * jnp.cos(thi), -w * jnp.sin(thi)], axis=1)


_tc_tables = pl.pallas_call(
    _tables_body,
    out_shape=[
        jax.ShapeDtypeStruct((2 * _F, _N), jnp.float32),
        jax.ShapeDtypeStruct((_N, 2 * _F), jnp.float32),
    ],
)


def _scan_body(w1_ref, w2_ref, cnt_ref, out_ref):
    la = jnp.dot(w1_ref[...], cnt_ref[...], preferred_element_type=jnp.float32)
    l = la[:_F, :]
    a = la[_F:, :]
    el = jnp.exp(l)
    s2 = jnp.concatenate([el * jnp.cos(a), el * jnp.sin(a)], axis=0)
    s = jnp.dot(w2_ref[...], s2, preferred_element_type=jnp.float32)
    out_ref[...] = jnp.log(jnp.maximum(s, 1e-9))


_tc_scan = pl.pallas_call(
    _scan_body,
    grid=(_B // _BBLK,),
    in_specs=[
        pl.BlockSpec((2 * _F, _N), lambda i: (0, 0)),
        pl.BlockSpec((_N, 2 * _F), lambda i: (0, 0)),
        pl.BlockSpec((_N, _BBLK), lambda i: (0, i)),
    ],
    out_specs=pl.BlockSpec((_N, _BBLK), lambda i: (0, i)),
    out_shape=jax.ShapeDtypeStruct((_N, _B), jnp.float32),
)


def kernel(route_logits, input_ids, mul):
    del mul
    counts_t = _sc_histogram()(jnp.transpose(input_ids))
    w1, w2 = _tc_tables(route_logits)
    return jnp.transpose(_tc_scan(w1, w2, counts_t))


# speedup vs baseline: 1.0375x; 1.0008x over previous
import functools
import math

import jax
import jax.numpy as jnp
from jax import lax
from jax.experimental import pallas as pl
from jax.experimental.pallas import tpu as pltpu
from jax.experimental.pallas import tpu_sc as plsc

_N = 60
_F = 32
_B = 4096
_T = 50
_NC, _NS = 2, 16
_NW = _NC * _NS
_CPW = _B // _NW
_LANES = 16
_NG = _CPW // _LANES
_BBLK = 2048


def _sc_histogram_body(ids_hbm, out_hbm, ids_v, cnt_v, in_sem):
    wid = lax.axis_index("s") * _NC + lax.axis_index("c")
    lanes = lax.broadcasted_iota(jnp.int32, (_LANES,), 0)
    zeros = jnp.zeros((_LANES,), jnp.float32)
    ones = jnp.ones((_LANES,), jnp.float32)
    col0 = wid * _CPW
    in_copy = pltpu.async_copy(ids_hbm.at[:, pl.ds(col0, _CPW)], ids_v, in_sem)

    @plsc.parallel_loop(0, _N, step=1, unroll=2)
    def _zero_row(r):
        for c in range(0, _CPW, _LANES):
            cnt_v[r, pl.ds(c, _LANES)] = zeros

    in_copy.wait()

    @plsc.parallel_loop(0, _T, step=1, unroll=2)
    def _step(t):
        tv = lanes * 0 + t
        for g in range(_NG):
            cols = g * _LANES + lanes
            tok = plsc.load_gather(ids_v, [tv, cols])
            plsc.addupdate_scatter(cnt_v, [tok, cols], ones)

    pltpu.sync_copy(cnt_v, out_hbm.at[:, pl.ds(col0, _CPW)])


@functools.cache
def _sc_histogram():
    mesh = plsc.VectorSubcoreMesh(
        core_axis_name="c", subcore_axis_name="s", num_cores=_NC, num_subcores=_NS
    )
    return pl.kernel(
        _sc_histogram_body,
        out_type=jax.ShapeDtypeStruct((_N, _B), jnp.float32),
        mesh=mesh,
        scratch_types=[
            pltpu.VMEM((_T, _CPW), jnp.int32),
            pltpu.VMEM((_N, _CPW), jnp.float32),
            pltpu.SemaphoreType.DMA,
        ],
        compiler_params=pltpu.CompilerParams(
            needs_layout_passes=False, disable_bounds_checks=True,
            skip_device_barrier=True),
    )


def _tables_body(rl_ref, w1_ref, w2_ref):
    rlt = jnp.transpose(rl_ref[...])
    m = jnp.max(rlt, axis=0, keepdims=True)
    e = jnp.exp(rlt - m)
    pt = e / jnp.sum(e, axis=0, keepdims=True)
    fi = lax.broadcasted_iota(jnp.int32, (_F, _N), 0)
    ki = lax.broadcasted_iota(jnp.int32, (_F, _N), 1)
    thf = ((fi * ki) % _N).astype(jnp.float32) * (2.0 * math.pi / _N)
    re = jnp.dot(jnp.cos(thf), pt, preferred_element_type=jnp.float32,
                 precision=lax.Precision.HIGHEST)
    im = -jnp.dot(jnp.sin(thf), pt, preferred_element_type=jnp.float32,
                  precision=lax.Precision.HIGHEST)
    live = fi < (_N // 2 + 1)
    lam = jnp.where(
        live, 0.5 * jnp.log(jnp.maximum(re * re + im * im, 1e-30)), 0.0)
    alp = jnp.where(live, jnp.arctan2(im, re), 0.0)
    w1_ref[...] = jnp.concatenate([lam, alp], axis=0)
    mi = lax.broadcasted_iota(jnp.int32, (_N, _F), 0)
    fj = lax.broadcasted_iota(jnp.int32, (_N, _F), 1)
    thi = ((mi * fj) % _N).astype(jnp.float32) * (2.0 * math.pi / _N)
    w = jnp.where((fj == 0) | (fj == _N // 2), 1.0, 2.0) * (1.0 / _N)
    w = jnp.where(fj < (_N // 2 + 1), w, 0.0)
    w2_ref[...] = jnp.concatenate([w * jnp.cos(thi), -w * jnp.sin(thi)], axis=1)


_tc_tables = pl.pallas_call(
    _tables_body,
    out_shape=[
        jax.ShapeDtypeStruct((2 * _F, _N), jnp.float32),
        jax.ShapeDtypeStruct((_N, 2 * _F), jnp.float32),
    ],
)


def _scan_body(w1_ref, w2_ref, cnt_ref, out_ref):
    la = jnp.dot(w1_ref[...], cnt_ref[...], preferred_element_type=jnp.float32)
    l = la[:_F, :]
    a = la[_F:, :]
    el = jnp.exp(l)
    s2 = jnp.concatenate([el * jnp.cos(a), el * jnp.sin(a)], axis=0)
    s = jnp.dot(w2_ref[...], s2, preferred_element_type=jnp.float32)
    out_ref[...] = jnp.log(jnp.maximum(s, 1e-9))


_tc_scan = pl.pallas_call(
    _scan_body,
    grid=(_B // _BBLK,),
    in_specs=[
        pl.BlockSpec((2 * _F, _N), lambda i: (0, 0)),
        pl.BlockSpec((_N, 2 * _F), lambda i: (0, 0)),
        pl.BlockSpec((_N, _BBLK), lambda i: (0, i)),
    ],
    out_specs=pl.BlockSpec((_N, _BBLK), lambda i: (0, i)),
    out_shape=jax.ShapeDtypeStruct((_N, _B), jnp.float32),
)


def kernel(route_logits, input_ids, mul):
    del mul
    counts_t = _sc_histogram()(jnp.transpose(input_ids))
    w1, w2 = _tc_tables(route_logits)
    return jnp.transpose(_tc_scan(w1, w2, counts_t))
